# baseline (speedup 1.0000x reference)
import functools

import jax
import jax.numpy as jnp
from jax import lax
from jax.experimental import pallas as pl
from jax.experimental.pallas import tpu as pltpu
from jax.experimental.pallas import tpu_sc as plsc

M_ROWS = 1000000
D = 64
B = 16384
NS = 16
CHUNK = B // NS
K = 128
NK = CHUNK // K
EPS = 1e-6

_SC_MESH = plsc.VectorSubcoreMesh(
    core_axis_name="c", subcore_axis_name="s", num_cores=1, num_subcores=NS)
_SC_PARAMS = pltpu.CompilerParams(use_tc_tiling_on_sc=False)


D2 = 2 * D


def _sc_gather_body(idx2d, bids2d, mem_val2, mem_state,
                    rows_out, stg_out, owner_out,
                    idx_v, idxh_v, b_v, rows_v, st_v, sem_s, sem_r, sem_t):
    tile = lax.axis_index("s")
    base = tile * CHUNK
    pltpu.sync_copy(idx2d.at[pl.ds(tile * NK, NK)], idx_v)
    pltpu.sync_copy(bids2d.at[pl.ds(tile * NK, NK)], b_v)
    for j in range(NK):
        for c in range(K // 16):
            idxh_v[j, pl.ds(c * 16, 16)] = (
                idx_v[j, pl.ds(c * 16, 16)] >> 1)
    cps = []
    for j in range(NK):
        cps.append(pltpu.async_copy(b_v.at[j], owner_out.at[idx_v.at[j]],
                                    sem_s))
        cps.append(pltpu.async_copy(mem_state.at[idx_v.at[j]],
                                    st_v.at[pl.ds(j * K, K)], sem_t))
    half = NK // 2
    for g in range(2):
        gcps = [pltpu.async_copy(mem_val2.at[idxh_v.at[g * half + j]],
                                 rows_v.at[pl.ds(j * K, K)], sem_r)
                for j in range(half)]
        for cp in gcps:
            cp.wait()
        pltpu.sync_copy(rows_v,
                        rows_out.at[pl.ds(base + g * half * K, half * K)])
    for cp in cps:
        cp.wait()
    pltpu.sync_copy(st_v, stg_out.at[pl.ds(base, CHUNK)])


_sc_gather = pl.kernel(
    _sc_gather_body,
    out_type=(jax.ShapeDtypeStruct((B, D2), jnp.float32),
              jax.ShapeDtypeStruct((B,), jnp.float32),
              jax.ShapeDtypeStruct((M_ROWS,), jnp.int32)),
    mesh=_SC_MESH,
    scratch_types=[
        pltpu.VMEM((NK, K), jnp.int32),
        pltpu.VMEM((NK, K), jnp.int32),
        pltpu.VMEM((NK, K), jnp.int32),
        pltpu.VMEM((CHUNK // 2, D2), jnp.float32),
        pltpu.VMEM((CHUNK,), jnp.float32),
        pltpu.SemaphoreType.DMA,
        pltpu.SemaphoreType.DMA,
        pltpu.SemaphoreType.DMA,
    ],
    compiler_params=pltpu.CompilerParams(use_tc_tiling_on_sc=True),
)


DH = D // 2


def _sc_segsum_body(idx2d, owner, c_lo, c_hi, gate, zeros2d, zeros1d,
                    sv_lo, sv_hi, segst_out,
                    idx_v, w_v, c_v, g_v, sem_w, sem_a, sem_g,
                    acc_sh, st_sh):
    tile = lax.axis_index("s")
    base = tile * CHUNK
    pltpu.sync_copy(idx2d.at[pl.ds(tile * NK, NK)], idx_v)
    cps = [pltpu.async_copy(owner.at[idx_v.at[j]], w_v.at[j], sem_w)
           for j in range(NK)]
    cps.append(pltpu.async_copy(gate.at[pl.ds(base, CHUNK)], g_v, sem_g))
    for cp in cps:
        cp.wait()
    for half, (src, dst) in enumerate(((c_lo, sv_lo), (c_hi, sv_hi))):
        pltpu.sync_copy(zeros2d, acc_sh.at[pl.ds(base, CHUNK)])
        if half == 0:
            pltpu.sync_copy(zeros1d, st_sh.at[pl.ds(base, CHUNK)])
        pltpu.sync_copy(src.at[pl.ds(base, CHUNK)], c_v)
        plsc.subcore_barrier()
        for j in range(NK):
            pltpu.sync_copy(c_v.at[pl.ds(j * K, K)], acc_sh.at[w_v.at[j]],
                            add=True)
            if half == 0:
                pltpu.sync_copy(g_v.at[pl.ds(j * K, K)], st_sh.at[w_v.at[j]],
                                add=True)
        plsc.subcore_barrier()
        cps = []
        for j in range(NK):
            cps.append(pltpu.async_copy(acc_sh.at[w_v.at[j]],
                                        c_v.at[pl.ds(j * K, K)], sem_a))
            if half == 0:
                cps.append(pltpu.async_copy(st_sh.at[w_v.at[j]],
                                            g_v.at[pl.ds(j * K, K)], sem_g))
        for cp in cps:
            cp.wait()
        pltpu.sync_copy(c_v, dst.at[pl.ds(base, CHUNK)])
        if half == 0:
            pltpu.sync_copy(g_v, segst_out.at[pl.ds(base, CHUNK)])
        plsc.subcore_barrier()


_sc_segsum = pl.kernel(
    _sc_segsum_body,
    out_type=(jax.ShapeDtypeStruct((B, DH), jnp.float32),
              jax.ShapeDtypeStruct((B, DH), jnp.float32),
              jax.ShapeDtypeStruct((B,), jnp.float32)),
    mesh=_SC_MESH,
    scratch_types=[
        pltpu.VMEM((NK, K), jnp.int32),
        pltpu.VMEM((NK, K), jnp.int32),
        pltpu.VMEM((CHUNK, DH), jnp.float32),
        pltpu.VMEM((CHUNK,), jnp.float32),
        pltpu.SemaphoreType.DMA,
        pltpu.SemaphoreType.DMA,
        pltpu.SemaphoreType.DMA,
        pltpu.VMEM_SHARED((B, DH), jnp.float32),
        pltpu.VMEM_SHARED((B,), jnp.float32),
    ],
    compiler_params=_SC_PARAMS,
)


def _tc_route_body(val_ref, rows_ref, par_ref, gn_ref, clo_ref, chi_ref,
                   gate_ref):
    v = val_ref[...]
    rp = rows_ref[...]
    odd = (par_ref[...] & 1) == 1
    r = jnp.where(odd, rp[:, D:], rp[:, :D])
    vn = v / (jnp.sqrt(jnp.sum(v * v, axis=-1, keepdims=True)) + EPS)
    gn = r / (jnp.sqrt(jnp.sum(r * r, axis=-1, keepdims=True)) + EPS)
    route = jnp.sum(gn * vn, axis=-1, keepdims=True)
    gate = jax.nn.softplus(route)
    contrib = gate * vn
    gn_ref[...] = gn
    clo_ref[...] = contrib[:, :DH]
    chi_ref[...] = contrib[:, DH:]
    gate_ref[...] = gate


def _tc_finish_body(gn_ref, svlo_ref, svhi_ref, stg_ref, segst_ref, out_ref):
    sv = jnp.concatenate([svlo_ref[...], svhi_ref[...]], axis=-1)
    nv = gn_ref[...] + sv
    nvn = nv / (jnp.sqrt(jnp.sum(nv * nv, axis=-1, keepdims=True)) + EPS)
    out_ref[...] = nvn * jnp.tanh(stg_ref[...] + segst_ref[...])


_TC_BLK = 2048
_TC_GRID = B // _TC_BLK
_row_spec = pl.BlockSpec((_TC_BLK, D), lambda i: (i, 0))
_wide_spec = pl.BlockSpec((_TC_BLK, D2), lambda i: (i, 0))
_half_spec = pl.BlockSpec((_TC_BLK, DH), lambda i: (i, 0))
_col_spec = pl.BlockSpec((_TC_BLK, 1), lambda i: (i, 0))

_tc_route = pl.pallas_call(
    _tc_route_body,
    grid=(_TC_GRID,),
    in_specs=[_row_spec, _wide_spec, _col_spec],
    out_specs=[_row_spec, _half_spec, _half_spec, _col_spec],
    out_shape=(jax.ShapeDtypeStruct((B, D), jnp.float32),
               jax.ShapeDtypeStruct((B, DH), jnp.float32),
               jax.ShapeDtypeStruct((B, DH), jnp.float32),
               jax.ShapeDtypeStruct((B, 1), jnp.float32)),
)

_tc_finish = pl.pallas_call(
    _tc_finish_body,
    grid=(_TC_GRID,),
    in_specs=[_row_spec, _half_spec, _half_spec, _col_spec, _col_spec],
    out_specs=_row_spec,
    out_shape=jax.ShapeDtypeStruct((B, D), jnp.float32),
)


@jax.jit
def kernel(mem_state, mem_val, val, idx):
    idx32 = idx.astype(jnp.int32)
    idx2d = idx32.reshape(B // K, K)
    bids2d = jnp.arange(B, dtype=jnp.int32).reshape(B // K, K)
    zeros2d = jnp.zeros((CHUNK, DH), jnp.float32)
    zeros1d = jnp.zeros((CHUNK,), jnp.float32)
    mem_val2 = mem_val.reshape(M_ROWS // 2, D2)

    rows, stg, owner = _sc_gather(idx2d, bids2d, mem_val2, mem_state)
    gn, c_lo, c_hi, gate = _tc_route(val, rows, idx32.reshape(B, 1))
    sv_lo, sv_hi, segst = _sc_segsum(idx2d, owner, c_lo, c_hi,
                                     gate.reshape(B), zeros2d, zeros1d)
    return _tc_finish(gn, sv_lo, sv_hi, stg.reshape(B, 1),
                      segst.reshape(B, 1))


# speedup vs baseline: 1.5457x; 1.5457x over previous
import functools

import jax
import jax.numpy as jnp
from jax import lax
from jax.experimental import pallas as pl
from jax.experimental.pallas import tpu as pltpu
from jax.experimental.pallas import tpu_sc as plsc

M_ROWS = 1000000
D = 64
B = 16384
NS = 16
CHUNK = B // NS
K = 128
NK = CHUNK // K
EPS = 1e-6

_SC_MESH = plsc.VectorSubcoreMesh(
    core_axis_name="c", subcore_axis_name="s", num_cores=1, num_subcores=NS)
_SC_PARAMS = pltpu.CompilerParams(use_tc_tiling_on_sc=False)


NW2 = 2 * NS
CH2 = B // NW2
NK2 = CH2 // K
GRP = 64

_SC_MESH2 = plsc.VectorSubcoreMesh(
    core_axis_name="c", subcore_axis_name="s", num_cores=2, num_subcores=NS)


def _sc_gather_body(idx2d, bids2d, mem_val, mem_state,
                    rows_out, stg_out, owner_out,
                    idxs_v, b_v, rows_v, st_v, sem_s, sem_r, sem_t):
    wid = lax.axis_index("s") * 2 + lax.axis_index("c")
    base = wid * CH2
    pltpu.sync_copy(idx2d.at[pl.ds(wid * NK2, NK2)], idxs_v)
    pltpu.sync_copy(bids2d.at[pl.ds(wid * NK2, NK2)], b_v)
    cps = []
    for j in range(NK2):
        cps.append(pltpu.async_copy(b_v.at[j], owner_out.at[idxs_v.at[j]],
                                    sem_s))
        cps.append(pltpu.async_copy(mem_state.at[idxs_v.at[j]],
                                    st_v.at[pl.ds(j * K, K)], sem_t))
    prev = None
    for g in range(CH2 // GRP):
        cur = []
        for c in range(GRP // 16):
            i0 = g * GRP + c * 16
            vec = idxs_v[i0 // K, pl.ds(i0 % K, 16)]
            for l in range(16):
                i = i0 + l
                cur.append(pltpu.async_copy(
                    mem_val.at[pl.ds(vec[l], 1), :],
                    rows_v.at[pl.ds(i, 1), :], sem_r))
        if prev is not None:
            for cp in prev:
                cp.wait()
        prev = cur
    for cp in prev:
        cp.wait()
    for cp in cps:
        cp.wait()
    pltpu.sync_copy(rows_v, rows_out.at[pl.ds(base, CH2)])
    pltpu.sync_copy(st_v, stg_out.at[pl.ds(base, CH2)])


_sc_gather = pl.kernel(
    _sc_gather_body,
    out_type=(jax.ShapeDtypeStruct((B, D), jnp.float32),
              jax.ShapeDtypeStruct((B,), jnp.float32),
              jax.ShapeDtypeStruct((M_ROWS,), jnp.int32)),
    mesh=_SC_MESH2,
    scratch_types=[
        pltpu.VMEM((NK2, K), jnp.int32),
        pltpu.VMEM((NK2, K), jnp.int32),
        pltpu.VMEM((CH2, D), jnp.float32),
        pltpu.VMEM((CH2,), jnp.float32),
        pltpu.SemaphoreType.DMA,
        pltpu.SemaphoreType.DMA,
        pltpu.SemaphoreType.DMA,
    ],
    compiler_params=pltpu.CompilerParams(use_tc_tiling_on_sc=True),
)


DH = D // 2


def _sc_segsum_body(idx2d, owner, c_lo, c_hi, gate, zeros2d, zeros1d,
                    sv_lo, sv_hi, segst_out,
                    idx_v, w_v, c_v, g_v, sem_w, sem_a, sem_g,
                    acc_sh, st_sh):
    tile = lax.axis_index("s")
    base = tile * CHUNK
    pltpu.sync_copy(idx2d.at[pl.ds(tile * NK, NK)], idx_v)
    cps = [pltpu.async_copy(owner.at[idx_v.at[j]], w_v.at[j], sem_w)
           for j in range(NK)]
    cps.append(pltpu.async_copy(gate.at[pl.ds(base, CHUNK)], g_v, sem_g))
    for cp in cps:
        cp.wait()
    for half, (src, dst) in enumerate(((c_lo, sv_lo), (c_hi, sv_hi))):
        pltpu.sync_copy(zeros2d, acc_sh.at[pl.ds(base, CHUNK)])
        if half == 0:
            pltpu.sync_copy(zeros1d, st_sh.at[pl.ds(base, CHUNK)])
        pltpu.sync_copy(src.at[pl.ds(base, CHUNK)], c_v)
        plsc.subcore_barrier()
        for j in range(NK):
            pltpu.sync_copy(c_v.at[pl.ds(j * K, K)], acc_sh.at[w_v.at[j]],
                            add=True)
            if half == 0:
                pltpu.sync_copy(g_v.at[pl.ds(j * K, K)], st_sh.at[w_v.at[j]],
                                add=True)
        plsc.subcore_barrier()
        cps = []
        for j in range(NK):
            cps.append(pltpu.async_copy(acc_sh.at[w_v.at[j]],
                                        c_v.at[pl.ds(j * K, K)], sem_a))
            if half == 0:
                cps.append(pltpu.async_copy(st_sh.at[w_v.at[j]],
                                            g_v.at[pl.ds(j * K, K)], sem_g))
        for cp in cps:
            cp.wait()
        pltpu.sync_copy(c_v, dst.at[pl.ds(base, CHUNK)])
        if half == 0:
            pltpu.sync_copy(g_v, segst_out.at[pl.ds(base, CHUNK)])
        plsc.subcore_barrier()


_sc_segsum = pl.kernel(
    _sc_segsum_body,
    out_type=(jax.ShapeDtypeStruct((B, DH), jnp.float32),
              jax.ShapeDtypeStruct((B, DH), jnp.float32),
              jax.ShapeDtypeStruct((B,), jnp.float32)),
    mesh=_SC_MESH,
    scratch_types=[
        pltpu.VMEM((NK, K), jnp.int32),
        pltpu.VMEM((NK, K), jnp.int32),
        pltpu.VMEM((CHUNK, DH), jnp.float32),
        pltpu.VMEM((CHUNK,), jnp.float32),
        pltpu.SemaphoreType.DMA,
        pltpu.SemaphoreType.DMA,
        pltpu.SemaphoreType.DMA,
        pltpu.VMEM_SHARED((B, DH), jnp.float32),
        pltpu.VMEM_SHARED((B,), jnp.float32),
    ],
    compiler_params=_SC_PARAMS,
)


def _tc_route_body(val_ref, rows_ref, gn_ref, clo_ref, chi_ref, gate_ref):
    v = val_ref[...]
    r = rows_ref[...]
    vn = v / (jnp.sqrt(jnp.sum(v * v, axis=-1, keepdims=True)) + EPS)
    gn = r / (jnp.sqrt(jnp.sum(r * r, axis=-1, keepdims=True)) + EPS)
    route = jnp.sum(gn * vn, axis=-1, keepdims=True)
    gate = jax.nn.softplus(route)
    contrib = gate * vn
    gn_ref[...] = gn
    clo_ref[...] = contrib[:, :DH]
    chi_ref[...] = contrib[:, DH:]
    gate_ref[...] = gate


def _tc_finish_body(gn_ref, svlo_ref, svhi_ref, stg_ref, segst_ref, out_ref):
    sv = jnp.concatenate([svlo_ref[...], svhi_ref[...]], axis=-1)
    nv = gn_ref[...] + sv
    nvn = nv / (jnp.sqrt(jnp.sum(nv * nv, axis=-1, keepdims=True)) + EPS)
    out_ref[...] = nvn * jnp.tanh(stg_ref[...] + segst_ref[...])


_TC_BLK = 2048
_TC_GRID = B // _TC_BLK
_row_spec = pl.BlockSpec((_TC_BLK, D), lambda i: (i, 0))
_half_spec = pl.BlockSpec((_TC_BLK, DH), lambda i: (i, 0))
_col_spec = pl.BlockSpec((_TC_BLK, 1), lambda i: (i, 0))

_tc_route = pl.pallas_call(
    _tc_route_body,
    grid=(_TC_GRID,),
    in_specs=[_row_spec, _row_spec],
    out_specs=[_row_spec, _half_spec, _half_spec, _col_spec],
    out_shape=(jax.ShapeDtypeStruct((B, D), jnp.float32),
               jax.ShapeDtypeStruct((B, DH), jnp.float32),
               jax.ShapeDtypeStruct((B, DH), jnp.float32),
               jax.ShapeDtypeStruct((B, 1), jnp.float32)),
)

_tc_finish = pl.pallas_call(
    _tc_finish_body,
    grid=(_TC_GRID,),
    in_specs=[_row_spec, _half_spec, _half_spec, _col_spec, _col_spec],
    out_specs=_row_spec,
    out_shape=jax.ShapeDtypeStruct((B, D), jnp.float32),
)


@jax.jit
def kernel(mem_state, mem_val, val, idx):
    idx32 = idx.astype(jnp.int32)
    idx2d = idx32.reshape(B // K, K)
    bids2d = jnp.arange(B, dtype=jnp.int32).reshape(B // K, K)
    zeros2d = jnp.zeros((CHUNK, DH), jnp.float32)
    zeros1d = jnp.zeros((CHUNK,), jnp.float32)

    rows, stg, owner = _sc_gather(idx2d, bids2d, mem_val, mem_state)
    gn, c_lo, c_hi, gate = _tc_route(val, rows)
    sv_lo, sv_hi, segst = _sc_segsum(idx2d, owner, c_lo, c_hi,
                                     gate.reshape(B), zeros2d, zeros1d)
    return _tc_finish(gn, sv_lo, sv_hi, stg.reshape(B, 1),
                      segst.reshape(B, 1))


# speedup vs baseline: 1.5781x; 1.0210x over previous
import functools

import jax
import jax.numpy as jnp
from jax import lax
from jax.experimental import pallas as pl
from jax.experimental.pallas import tpu as pltpu
from jax.experimental.pallas import tpu_sc as plsc

M_ROWS = 1000000
D = 64
B = 16384
NS = 16
CHUNK = B // NS
K = 128
NK = CHUNK // K
EPS = 1e-6

_SC_MESH = plsc.VectorSubcoreMesh(
    core_axis_name="c", subcore_axis_name="s", num_cores=1, num_subcores=NS)
_SC_PARAMS = pltpu.CompilerParams(use_tc_tiling_on_sc=False)


NW2 = 2 * NS
CH2 = B // NW2
NK2 = CH2 // K
GRP = 64

_SC_MESH2 = plsc.VectorSubcoreMesh(
    core_axis_name="c", subcore_axis_name="s", num_cores=2, num_subcores=NS)


def _sc_prep_body(idx2d, bids2d, mem_state,
                  stg_out, owner_out,
                  idxs_v, b_v, st_v, sem_s, sem_t):
    wid = lax.axis_index("s") * 2 + lax.axis_index("c")
    base = wid * CH2
    pltpu.sync_copy(idx2d.at[pl.ds(wid * NK2, NK2)], idxs_v)
    pltpu.sync_copy(bids2d.at[pl.ds(wid * NK2, NK2)], b_v)
    cps = []
    for j in range(NK2):
        cps.append(pltpu.async_copy(b_v.at[j], owner_out.at[idxs_v.at[j]],
                                    sem_s))
        cps.append(pltpu.async_copy(mem_state.at[idxs_v.at[j]],
                                    st_v.at[pl.ds(j * K, K)], sem_t))
    for cp in cps:
        cp.wait()
    pltpu.sync_copy(st_v, stg_out.at[pl.ds(base, CH2)])


_sc_prep = pl.kernel(
    _sc_prep_body,
    out_type=(jax.ShapeDtypeStruct((B,), jnp.float32),
              jax.ShapeDtypeStruct((M_ROWS,), jnp.int32)),
    mesh=_SC_MESH2,
    scratch_types=[
        pltpu.VMEM((NK2, K), jnp.int32),
        pltpu.VMEM((NK2, K), jnp.int32),
        pltpu.VMEM((CH2,), jnp.float32),
        pltpu.SemaphoreType.DMA,
        pltpu.SemaphoreType.DMA,
    ],
    compiler_params=pltpu.CompilerParams(use_tc_tiling_on_sc=True),
)


def _sc_gather_body(idx2d, mem_val,
                    rows_out,
                    idxs_v, rows_v, sem_r):
    wid = lax.axis_index("s") * 2 + lax.axis_index("c")
    base = wid * CH2
    pltpu.sync_copy(idx2d.at[pl.ds(wid * NK2, NK2)], idxs_v)
    prev = None
    for g in range(CH2 // GRP):
        cur = []
        for c in range(GRP // 16):
            i0 = g * GRP + c * 16
            vec = idxs_v[i0 // K, pl.ds(i0 % K, 16)]
            for l in range(16):
                i = i0 + l
                cur.append(pltpu.async_copy(
                    mem_val.at[pl.ds(vec[l], 1), :],
                    rows_v.at[pl.ds(i, 1), :], sem_r))
        if prev is not None:
            for cp in prev:
                cp.wait()
        prev = cur
    for cp in prev:
        cp.wait()
    pltpu.sync_copy(rows_v, rows_out.at[pl.ds(base, CH2)])


_sc_gather = pl.kernel(
    _sc_gather_body,
    out_type=jax.ShapeDtypeStruct((B, D), jnp.float32),
    mesh=_SC_MESH2,
    scratch_types=[
        pltpu.VMEM((NK2, K), jnp.int32),
        pltpu.VMEM((CH2, D), jnp.float32),
        pltpu.SemaphoreType.DMA,
    ],
    compiler_params=pltpu.CompilerParams(use_tc_tiling_on_sc=True),
)


DH = D // 2


def _sc_segsum_body(idx2d, owner, c_lo, c_hi, gate, zeros2d, zeros1d,
                    sv_lo, sv_hi, segst_out,
                    idx_v, w_v, c_v, g_v, sem_w, sem_a, sem_g,
                    acc_sh, st_sh):
    tile = lax.axis_index("s")
    base = tile * CHUNK
    pltpu.sync_copy(idx2d.at[pl.ds(tile * NK, NK)], idx_v)
    cps = [pltpu.async_copy(owner.at[idx_v.at[j]], w_v.at[j], sem_w)
           for j in range(NK)]
    cps.append(pltpu.async_copy(gate.at[pl.ds(base, CHUNK)], g_v, sem_g))
    for cp in cps:
        cp.wait()
    for half, (src, dst) in enumerate(((c_lo, sv_lo), (c_hi, sv_hi))):
        pltpu.sync_copy(zeros2d, acc_sh.at[pl.ds(base, CHUNK)])
        if half == 0:
            pltpu.sync_copy(zeros1d, st_sh.at[pl.ds(base, CHUNK)])
        pltpu.sync_copy(src.at[pl.ds(base, CHUNK)], c_v)
        plsc.subcore_barrier()
        for j in range(NK):
            pltpu.sync_copy(c_v.at[pl.ds(j * K, K)], acc_sh.at[w_v.at[j]],
                            add=True)
            if half == 0:
                pltpu.sync_copy(g_v.at[pl.ds(j * K, K)], st_sh.at[w_v.at[j]],
                                add=True)
        plsc.subcore_barrier()
        cps = []
        for j in range(NK):
            cps.append(pltpu.async_copy(acc_sh.at[w_v.at[j]],
                                        c_v.at[pl.ds(j * K, K)], sem_a))
            if half == 0:
                cps.append(pltpu.async_copy(st_sh.at[w_v.at[j]],
                                            g_v.at[pl.ds(j * K, K)], sem_g))
        for cp in cps:
            cp.wait()
        pltpu.sync_copy(c_v, dst.at[pl.ds(base, CHUNK)])
        if half == 0:
            pltpu.sync_copy(g_v, segst_out.at[pl.ds(base, CHUNK)])
        plsc.subcore_barrier()


_sc_segsum = pl.kernel(
    _sc_segsum_body,
    out_type=(jax.ShapeDtypeStruct((B, DH), jnp.float32),
              jax.ShapeDtypeStruct((B, DH), jnp.float32),
              jax.ShapeDtypeStruct((B,), jnp.float32)),
    mesh=_SC_MESH,
    scratch_types=[
        pltpu.VMEM((NK, K), jnp.int32),
        pltpu.VMEM((NK, K), jnp.int32),
        pltpu.VMEM((CHUNK, DH), jnp.float32),
        pltpu.VMEM((CHUNK,), jnp.float32),
        pltpu.SemaphoreType.DMA,
        pltpu.SemaphoreType.DMA,
        pltpu.SemaphoreType.DMA,
        pltpu.VMEM_SHARED((B, DH), jnp.float32),
        pltpu.VMEM_SHARED((B,), jnp.float32),
    ],
    compiler_params=_SC_PARAMS,
)


def _tc_route_body(val_ref, rows_ref, gn_ref, clo_ref, chi_ref, gate_ref):
    v = val_ref[...]
    r = rows_ref[...]
    vn = v / (jnp.sqrt(jnp.sum(v * v, axis=-1, keepdims=True)) + EPS)
    gn = r / (jnp.sqrt(jnp.sum(r * r, axis=-1, keepdims=True)) + EPS)
    route = jnp.sum(gn * vn, axis=-1, keepdims=True)
    gate = jax.nn.softplus(route)
    contrib = gate * vn
    gn_ref[...] = gn
    clo_ref[...] = contrib[:, :DH]
    chi_ref[...] = contrib[:, DH:]
    gate_ref[...] = gate


def _tc_finish_body(gn_ref, svlo_ref, svhi_ref, stg_ref, segst_ref, out_ref):
    sv = jnp.concatenate([svlo_ref[...], svhi_ref[...]], axis=-1)
    nv = gn_ref[...] + sv
    nvn = nv / (jnp.sqrt(jnp.sum(nv * nv, axis=-1, keepdims=True)) + EPS)
    out_ref[...] = nvn * jnp.tanh(stg_ref[...] + segst_ref[...])


_TC_BLK = 2048
_TC_GRID = B // _TC_BLK
_row_spec = pl.BlockSpec((_TC_BLK, D), lambda i: (i, 0))
_half_spec = pl.BlockSpec((_TC_BLK, DH), lambda i: (i, 0))
_col_spec = pl.BlockSpec((_TC_BLK, 1), lambda i: (i, 0))

_tc_route = pl.pallas_call(
    _tc_route_body,
    grid=(_TC_GRID,),
    in_specs=[_row_spec, _row_spec],
    out_specs=[_row_spec, _half_spec, _half_spec, _col_spec],
    out_shape=(jax.ShapeDtypeStruct((B, D), jnp.float32),
               jax.ShapeDtypeStruct((B, DH), jnp.float32),
               jax.ShapeDtypeStruct((B, DH), jnp.float32),
               jax.ShapeDtypeStruct((B, 1), jnp.float32)),
)

_tc_finish = pl.pallas_call(
    _tc_finish_body,
    grid=(_TC_GRID,),
    in_specs=[_row_spec, _half_spec, _half_spec, _col_spec, _col_spec],
    out_specs=_row_spec,
    out_shape=jax.ShapeDtypeStruct((B, D), jnp.float32),
)


@jax.jit
def kernel(mem_state, mem_val, val, idx):
    idx32 = idx.astype(jnp.int32)
    idx2d = idx32.reshape(B // K, K)
    bids2d = jnp.arange(B, dtype=jnp.int32).reshape(B // K, K)
    zeros2d = jnp.zeros((CHUNK, DH), jnp.float32)
    zeros1d = jnp.zeros((CHUNK,), jnp.float32)

    stg, owner = _sc_prep(idx2d, bids2d, mem_state)
    rows = _sc_gather(idx2d, mem_val)
    gn, c_lo, c_hi, gate = _tc_route(val, rows)
    sv_lo, sv_hi, segst = _sc_segsum(idx2d, owner, c_lo, c_hi,
                                     gate.reshape(B), zeros2d, zeros1d)
    return _tc_finish(gn, sv_lo, sv_hi, stg.reshape(B, 1),
                      segst.reshape(B, 1))
